# baseline (speedup 1.0000x reference)
import functools

import jax
import jax.numpy as jnp
from jax import lax
from jax.experimental import pallas as pl
from jax.experimental.pallas import tpu as pltpu
from jax.experimental.pallas import tpu_sc as plsc

N = 10000
E = 320000
NP = 10016
NC = 2
NS = 16
NW = NC * NS
CH = 88
NB = 4
NCH = 116
NT = NCH // NB
EPT = NW * NCH * CH
NCHD = 79
EPD = NW * NCHD * 128
BNS = 1.0 / (1.0 + 1e-5) ** 0.5
R = 1000

_mesh = plsc.VectorSubcoreMesh(core_axis_name="c", subcore_axis_name="s")
_sc_params = pltpu.CompilerParams(needs_layout_passes=False)


def _deg_body(dst_hbm, deg_hbm, dstbuf, degbuf):
    c = lax.axis_index("c")
    s = lax.axis_index("s")
    wid = c * NS + s
    pltpu.sync_copy(dst_hbm.at[wid], dstbuf)
    z = jnp.zeros((16,), jnp.float32)

    def zero(i, carry):
        degbuf[i, pl.ds(0, 16)] = z
        return carry

    lax.fori_loop(0, NP // 16, zero, 0)
    ones = jnp.ones((16,), jnp.float32)

    def body(g, carry):
        for k in range(8):
            idx = dstbuf[g, pl.ds(k * 16, 16)]
            plsc.addupdate_scatter(degbuf, [idx >> 4, idx & 15], ones)
        return carry

    lax.fori_loop(0, NCHD, body, 0)
    pltpu.sync_copy(degbuf, deg_hbm.at[wid])


_deg_call = pl.kernel(
    _deg_body,
    out_type=jax.ShapeDtypeStruct((NW, NP // 16, 16), jnp.float32),
    mesh=_mesh,
    compiler_params=_sc_params,
    scratch_types=[
        pltpu.VMEM((NCHD, 128), jnp.int32),
        pltpu.VMEM((NP // 16, 16), jnp.float32),
    ],
)


def _make_scatter(F):
    def body(mp_hbm, src_hbm, dst_hbm, out_hbm,
             idxbuf, rows0, rows1, rows2, rows3,
             acc, gsem0, gsem1, gsem2, gsem3, isem):
        c = lax.axis_index("c")
        s = lax.axis_index("s")
        wid = c * NS + s
        rows = (rows0, rows1, rows2, rows3)
        gsems = (gsem0, gsem1, gsem2, gsem3)

        z = jnp.zeros((16,), jnp.float32)

        def zero(i, carry):
            for k in range(F // 16):
                rows0[i, pl.ds(k * 16, 16)] = z
            return carry

        lax.fori_loop(0, CH, zero, 0)
        base = s * 632

        def _zero_slice(nrows):
            nf = nrows // CH
            rm = nrows - nf * CH
            for k in range(nf):
                pltpu.sync_copy(rows0, acc.at[pl.ds(base + k * CH, CH)])
            if rm:
                pltpu.sync_copy(rows0.at[pl.ds(0, rm)],
                                acc.at[pl.ds(base + nf * CH, rm)])

        @pl.when(s < 15)
        def _():
            _zero_slice(632)

        @pl.when(s == 15)
        def _():
            _zero_slice(536)

        plsc.subcore_barrier()

        pltpu.sync_copy(src_hbm.at[wid, 0], idxbuf.at[pl.ds(0, NB)])
        pltpu.sync_copy(dst_hbm.at[wid, 0], idxbuf.at[pl.ds(2 * NB, NB)])
        for j in range(NB - 1):
            pltpu.async_copy(mp_hbm.at[idxbuf.at[j]], rows[j], gsems[j])

        def macro(t, carry):
            slot = lax.rem(t, 2)
            nslot = 1 - slot
            nxt = t + 1

            @pl.when(nxt < NT)
            def _():
                pltpu.async_copy(src_hbm.at[wid, nxt],
                                 idxbuf.at[pl.ds(nslot * NB, NB)], isem)
                pltpu.async_copy(dst_hbm.at[wid, nxt],
                                 idxbuf.at[pl.ds(2 * NB + nslot * NB, NB)],
                                 isem)

            for j in range(NB):
                pltpu.make_async_copy(mp_hbm.at[idxbuf.at[slot * NB + j]],
                                      rows[j], gsems[j]).wait()
                if j == 0:
                    pltpu.async_copy(mp_hbm.at[idxbuf.at[slot * NB + NB - 1]],
                                     rows[NB - 1], gsems[NB - 1])
                elif j == 1:
                    @pl.when(nxt < NT)
                    def _():
                        pltpu.make_async_copy(
                            src_hbm.at[wid, nxt],
                            idxbuf.at[pl.ds(nslot * NB, NB)], isem).wait()
                        pltpu.make_async_copy(
                            dst_hbm.at[wid, nxt],
                            idxbuf.at[pl.ds(2 * NB + nslot * NB, NB)],
                            isem).wait()
                        pltpu.async_copy(mp_hbm.at[idxbuf.at[nslot * NB]],
                                         rows[0], gsems[0])
                else:
                    @pl.when(nxt < NT)
                    def _():
                        pltpu.async_copy(
                            mp_hbm.at[idxbuf.at[nslot * NB + j - 1]],
                            rows[j - 1], gsems[j - 1])
                pltpu.sync_copy(rows[j],
                                acc.at[idxbuf.at[2 * NB + slot * NB + j]],
                                add=True)
            return carry

        lax.fori_loop(0, NT, macro, 0)
        plsc.subcore_barrier()

        @pl.when(s < 15)
        def _():
            pltpu.sync_copy(acc.at[pl.ds(base, 632)],
                            out_hbm.at[c, pl.ds(base, 632)])

        @pl.when(s == 15)
        def _():
            pltpu.sync_copy(acc.at[pl.ds(base, 536)],
                            out_hbm.at[c, pl.ds(base, 536)])

    return pl.kernel(
        body,
        out_type=jax.ShapeDtypeStruct((NC, NP, F), jnp.float32),
        mesh=_mesh,
        compiler_params=_sc_params,
        scratch_types=[
            pltpu.VMEM((4 * NB, CH), jnp.int32),
            pltpu.VMEM((CH, F), jnp.float32),
            pltpu.VMEM((CH, F), jnp.float32),
            pltpu.VMEM((CH, F), jnp.float32),
            pltpu.VMEM((CH, F), jnp.float32),
            pltpu.VMEM_SHARED((NP, F), jnp.float32),
            pltpu.SemaphoreType.DMA,
            pltpu.SemaphoreType.DMA,
            pltpu.SemaphoreType.DMA,
            pltpu.SemaphoreType.DMA,
            pltpu.SemaphoreType.DMA,
        ],
    )


_scat128 = _make_scatter(128)


def _b1_body(parts_ref, x_ref, w_ref, mp_ref, dis_ref):
    dis = lax.rsqrt(1.0 + jnp.sum(parts_ref[...], axis=1))[:, None]
    dis_ref[...] = dis
    m = jnp.dot(x_ref[...], w_ref[...], preferred_element_type=jnp.float32)
    mp_ref[...] = m * dis


_b1_call = pl.pallas_call(
    _b1_body,
    grid=(N // R,),
    in_specs=[
        pl.BlockSpec((R, NW), lambda i: (i, 0)),
        pl.BlockSpec((R, 128), lambda i: (i, 0)),
        pl.BlockSpec((128, 128), lambda i: (0, 0)),
    ],
    out_specs=[
        pl.BlockSpec((R, 128), lambda i: (i, 0)),
        pl.BlockSpec((R, 1), lambda i: (i, 0)),
    ],
    out_shape=[
        jax.ShapeDtypeStruct((N, 128), jnp.float32),
        jax.ShapeDtypeStruct((N, 1), jnp.float32),
    ],
)


def _b2_body(dis_ref, acc_ref, mp1_ref, a1_ref, b1_ref, w2_ref,
             h1_ref, mp2_ref):
    sdis = dis_ref[...]
    conv = sdis * (acc_ref[0] + acc_ref[1] + mp1_ref[...])
    h1 = jnp.maximum(conv * a1_ref[...][None, :] + b1_ref[...][None, :], 0.0)
    h1_ref[...] = h1
    mp2_ref[...] = sdis * jnp.dot(h1, w2_ref[...],
                                  preferred_element_type=jnp.float32)


_b2_call = pl.pallas_call(
    _b2_body,
    grid=(N // R,),
    in_specs=[
        pl.BlockSpec((R, 1), lambda i: (i, 0)),
        pl.BlockSpec((NC, R, 128), lambda i: (0, i, 0)),
        pl.BlockSpec((R, 128), lambda i: (i, 0)),
        pl.BlockSpec((128,), lambda i: (0,)),
        pl.BlockSpec((128,), lambda i: (0,)),
        pl.BlockSpec((128, 128), lambda i: (0, 0)),
    ],
    out_specs=[
        pl.BlockSpec((R, 128), lambda i: (i, 0)),
        pl.BlockSpec((R, 128), lambda i: (i, 0)),
    ],
    out_shape=[
        jax.ShapeDtypeStruct((N, 128), jnp.float32),
        jax.ShapeDtypeStruct((N, 128), jnp.float32),
    ],
)


def _b3_body(dis_ref, acc_ref, mp2_ref, h1_ref, a2_ref, b2_ref,
             lng_ref, lnb_ref, wavg_ref, bavg_ref, w3_ref, mp3_ref):
    sdis = dis_ref[...]
    conv = sdis * (acc_ref[0] + acc_ref[1] + mp2_ref[...])
    h2 = jnp.maximum(conv * a2_ref[...][None, :] + b2_ref[...][None, :], 0.0)
    d = (h2 - h1_ref[...]) * 0.5
    mu = jnp.mean(d, axis=1, keepdims=True)
    dc = d - mu
    var = jnp.mean(dc * dc, axis=1, keepdims=True)
    z = dc / jnp.sqrt(var) * lng_ref[...][None, :] + lnb_ref[...][None, :]
    xr = h2 + jnp.dot(z, wavg_ref[...], preferred_element_type=jnp.float32) \
        + bavg_ref[...][None, :]
    mp3_ref[...] = sdis * jnp.dot(xr, w3_ref[...],
                                  preferred_element_type=jnp.float32)


_b3_call = pl.pallas_call(
    _b3_body,
    grid=(N // R,),
    in_specs=[
        pl.BlockSpec((R, 1), lambda i: (i, 0)),
        pl.BlockSpec((NC, R, 128), lambda i: (0, i, 0)),
        pl.BlockSpec((R, 128), lambda i: (i, 0)),
        pl.BlockSpec((R, 128), lambda i: (i, 0)),
        pl.BlockSpec((128,), lambda i: (0,)),
        pl.BlockSpec((128,), lambda i: (0,)),
        pl.BlockSpec((128,), lambda i: (0,)),
        pl.BlockSpec((128,), lambda i: (0,)),
        pl.BlockSpec((128, 128), lambda i: (0, 0)),
        pl.BlockSpec((128,), lambda i: (0,)),
        pl.BlockSpec((128, 128), lambda i: (0, 0)),
    ],
    out_specs=pl.BlockSpec((R, 128), lambda i: (i, 0)),
    out_shape=jax.ShapeDtypeStruct((N, 128), jnp.float32),
)


def _b4_body(dis_ref, acc_ref, mp3_ref, b3_ref, out_ref):
    sdis = dis_ref[...]
    out_ref[...] = sdis * (acc_ref[0][:, :64] + acc_ref[1][:, :64]
                           + mp3_ref[...][:, :64]) + b3_ref[...][None, :]


_b4_call = pl.pallas_call(
    _b4_body,
    grid=(N // R,),
    in_specs=[
        pl.BlockSpec((R, 1), lambda i: (i, 0)),
        pl.BlockSpec((NC, R, 128), lambda i: (0, i, 0)),
        pl.BlockSpec((R, 128), lambda i: (i, 0)),
        pl.BlockSpec((64,), lambda i: (0,)),
    ],
    out_specs=pl.BlockSpec((R, 64), lambda i: (i, 0)),
    out_shape=jax.ShapeDtypeStruct((N, 64), jnp.float32),
)


def kernel(x, adj, W1, b1, g1, be1, W2, b2, g2, be2,
           lng, lnb, Wavg, bavg, W3, b3):
    src = adj[0]
    dst = adj[1]
    npad = EPT - E
    ar = jnp.arange(npad, dtype=jnp.int32)
    srcp = jnp.concatenate([src, ar % N]).reshape(NW, NT, NB, CH)
    dstp_flat = jnp.concatenate([dst, N + ar % (NP - N)])
    dstp = dstp_flat.reshape(NW, NT, NB, CH)

    ard = jnp.arange(EPD - E, dtype=jnp.int32)
    dstp_deg = jnp.concatenate([dst, N + ard % (NP - N)]).reshape(
        NW, NCHD, 128)
    degparts = _deg_call(dstp_deg).reshape(NW, NP).T

    a1 = BNS * g1
    b1e = b1 * a1 + be1
    a2 = BNS * g2
    b2e = b2 * a2 + be2

    mp1, dis = _b1_call(degparts, x, W1)
    acc1 = _scat128(mp1, srcp, dstp)
    h1, mp2 = _b2_call(dis, acc1, mp1, a1, b1e, W2)
    acc2 = _scat128(mp2, srcp, dstp)
    W3p = jnp.pad(W3, ((0, 0), (0, 64)))
    mp3 = _b3_call(dis, acc2, mp2, h1, a2, b2e, lng, lnb, Wavg, bavg, W3p)
    acc3 = _scat128(mp3, srcp, dstp)
    out = _b4_call(dis, acc3, mp3, b3)
    return out


# speedup vs baseline: 1.0129x; 1.0129x over previous
import functools

import jax
import jax.numpy as jnp
from jax import lax
from jax.experimental import pallas as pl
from jax.experimental.pallas import tpu as pltpu
from jax.experimental.pallas import tpu_sc as plsc

N = 10000
E = 320000
NP = 10016
NC = 2
NS = 16
NW = NC * NS
CH = 120
NB = 3
NCH = 84
NT = NCH // NB
EPT = NW * NCH * CH
NCHD = 79
EPD = NW * NCHD * 128
BNS = 1.0 / (1.0 + 1e-5) ** 0.5
R = 1000

_mesh = plsc.VectorSubcoreMesh(core_axis_name="c", subcore_axis_name="s")
_sc_params = pltpu.CompilerParams(needs_layout_passes=False)


def _deg_body(dst_hbm, deg_hbm, dstbuf, degbuf):
    c = lax.axis_index("c")
    s = lax.axis_index("s")
    wid = c * NS + s
    pltpu.sync_copy(dst_hbm.at[wid], dstbuf)
    z = jnp.zeros((16,), jnp.float32)

    def zero(i, carry):
        degbuf[i, pl.ds(0, 16)] = z
        return carry

    lax.fori_loop(0, NP // 16, zero, 0)
    ones = jnp.ones((16,), jnp.float32)

    def body(g, carry):
        for k in range(8):
            idx = dstbuf[g, pl.ds(k * 16, 16)]
            plsc.addupdate_scatter(degbuf, [idx >> 4, idx & 15], ones)
        return carry

    lax.fori_loop(0, NCHD, body, 0)
    pltpu.sync_copy(degbuf, deg_hbm.at[wid])


_deg_call = pl.kernel(
    _deg_body,
    out_type=jax.ShapeDtypeStruct((NW, NP // 16, 16), jnp.float32),
    mesh=_mesh,
    compiler_params=_sc_params,
    scratch_types=[
        pltpu.VMEM((NCHD, 128), jnp.int32),
        pltpu.VMEM((NP // 16, 16), jnp.float32),
    ],
)


def _make_scatter(F):
    def body(mp_hbm, src_hbm, dst_hbm, out_hbm, idxbuf, *rest):
        rows = rest[:NB]
        acc = rest[NB]
        gsems = rest[NB + 1:2 * NB + 1]
        isem = rest[2 * NB + 1]
        rows0 = rows[0]
        c = lax.axis_index("c")
        s = lax.axis_index("s")
        wid = c * NS + s

        z = jnp.zeros((16,), jnp.float32)

        def zero(i, carry):
            for k in range(F // 16):
                rows0[i, pl.ds(k * 16, 16)] = z
            return carry

        lax.fori_loop(0, CH, zero, 0)
        base = s * 632

        def _zero_slice(nrows):
            nf = nrows // CH
            rm = nrows - nf * CH
            for k in range(nf):
                pltpu.sync_copy(rows0, acc.at[pl.ds(base + k * CH, CH)])
            if rm:
                pltpu.sync_copy(rows0.at[pl.ds(0, rm)],
                                acc.at[pl.ds(base + nf * CH, rm)])

        @pl.when(s < 15)
        def _():
            _zero_slice(632)

        @pl.when(s == 15)
        def _():
            _zero_slice(536)

        plsc.subcore_barrier()

        pltpu.sync_copy(src_hbm.at[wid, 0], idxbuf.at[pl.ds(0, NB)])
        pltpu.sync_copy(dst_hbm.at[wid, 0], idxbuf.at[pl.ds(2 * NB, NB)])
        for j in range(NB - 1):
            pltpu.async_copy(mp_hbm.at[idxbuf.at[j]], rows[j], gsems[j])

        def macro(t, carry):
            slot = lax.rem(t, 2)
            nslot = 1 - slot
            nxt = t + 1

            @pl.when(nxt < NT)
            def _():
                pltpu.async_copy(src_hbm.at[wid, nxt],
                                 idxbuf.at[pl.ds(nslot * NB, NB)], isem)
                pltpu.async_copy(dst_hbm.at[wid, nxt],
                                 idxbuf.at[pl.ds(2 * NB + nslot * NB, NB)],
                                 isem)

            for j in range(NB):
                pltpu.make_async_copy(mp_hbm.at[idxbuf.at[slot * NB + j]],
                                      rows[j], gsems[j]).wait()
                if j == 0:
                    pltpu.async_copy(mp_hbm.at[idxbuf.at[slot * NB + NB - 1]],
                                     rows[NB - 1], gsems[NB - 1])
                elif j == 1:
                    @pl.when(nxt < NT)
                    def _():
                        pltpu.make_async_copy(
                            src_hbm.at[wid, nxt],
                            idxbuf.at[pl.ds(nslot * NB, NB)], isem).wait()
                        pltpu.make_async_copy(
                            dst_hbm.at[wid, nxt],
                            idxbuf.at[pl.ds(2 * NB + nslot * NB, NB)],
                            isem).wait()
                        pltpu.async_copy(mp_hbm.at[idxbuf.at[nslot * NB]],
                                         rows[0], gsems[0])
                else:
                    @pl.when(nxt < NT)
                    def _():
                        pltpu.async_copy(
                            mp_hbm.at[idxbuf.at[nslot * NB + j - 1]],
                            rows[j - 1], gsems[j - 1])
                pltpu.sync_copy(rows[j],
                                acc.at[idxbuf.at[2 * NB + slot * NB + j]],
                                add=True)
            return carry

        lax.fori_loop(0, NT, macro, 0)
        plsc.subcore_barrier()

        @pl.when(s < 15)
        def _():
            pltpu.sync_copy(acc.at[pl.ds(base, 632)],
                            out_hbm.at[c, pl.ds(base, 632)])

        @pl.when(s == 15)
        def _():
            pltpu.sync_copy(acc.at[pl.ds(base, 536)],
                            out_hbm.at[c, pl.ds(base, 536)])

    return pl.kernel(
        body,
        out_type=jax.ShapeDtypeStruct((NC, NP, F), jnp.float32),
        mesh=_mesh,
        compiler_params=_sc_params,
        scratch_types=(
            [pltpu.VMEM((4 * NB, CH), jnp.int32)]
            + [pltpu.VMEM((CH, F), jnp.float32) for _ in range(NB)]
            + [pltpu.VMEM_SHARED((NP, F), jnp.float32)]
            + [pltpu.SemaphoreType.DMA for _ in range(NB + 1)]
        ),
    )


_scat128 = _make_scatter(128)


def _b1_body(parts_ref, x_ref, w_ref, mp_ref, dis_ref):
    dis = lax.rsqrt(1.0 + jnp.sum(parts_ref[...], axis=1))[:, None]
    dis_ref[...] = dis
    m = jnp.dot(x_ref[...], w_ref[...], preferred_element_type=jnp.float32)
    mp_ref[...] = m * dis


_b1_call = pl.pallas_call(
    _b1_body,
    grid=(N // R,),
    in_specs=[
        pl.BlockSpec((R, NW), lambda i: (i, 0)),
        pl.BlockSpec((R, 128), lambda i: (i, 0)),
        pl.BlockSpec((128, 128), lambda i: (0, 0)),
    ],
    out_specs=[
        pl.BlockSpec((R, 128), lambda i: (i, 0)),
        pl.BlockSpec((R, 1), lambda i: (i, 0)),
    ],
    out_shape=[
        jax.ShapeDtypeStruct((N, 128), jnp.float32),
        jax.ShapeDtypeStruct((N, 1), jnp.float32),
    ],
)


def _b2_body(dis_ref, acc_ref, mp1_ref, a1_ref, b1_ref, w2_ref,
             h1_ref, mp2_ref):
    sdis = dis_ref[...]
    conv = sdis * (acc_ref[0] + acc_ref[1] + mp1_ref[...])
    h1 = jnp.maximum(conv * a1_ref[...][None, :] + b1_ref[...][None, :], 0.0)
    h1_ref[...] = h1
    mp2_ref[...] = sdis * jnp.dot(h1, w2_ref[...],
                                  preferred_element_type=jnp.float32)


_b2_call = pl.pallas_call(
    _b2_body,
    grid=(N // R,),
    in_specs=[
        pl.BlockSpec((R, 1), lambda i: (i, 0)),
        pl.BlockSpec((NC, R, 128), lambda i: (0, i, 0)),
        pl.BlockSpec((R, 128), lambda i: (i, 0)),
        pl.BlockSpec((128,), lambda i: (0,)),
        pl.BlockSpec((128,), lambda i: (0,)),
        pl.BlockSpec((128, 128), lambda i: (0, 0)),
    ],
    out_specs=[
        pl.BlockSpec((R, 128), lambda i: (i, 0)),
        pl.BlockSpec((R, 128), lambda i: (i, 0)),
    ],
    out_shape=[
        jax.ShapeDtypeStruct((N, 128), jnp.float32),
        jax.ShapeDtypeStruct((N, 128), jnp.float32),
    ],
)


def _b3_body(dis_ref, acc_ref, mp2_ref, h1_ref, a2_ref, b2_ref,
             lng_ref, lnb_ref, wavg_ref, bavg_ref, w3_ref, mp3_ref):
    sdis = dis_ref[...]
    conv = sdis * (acc_ref[0] + acc_ref[1] + mp2_ref[...])
    h2 = jnp.maximum(conv * a2_ref[...][None, :] + b2_ref[...][None, :], 0.0)
    d = (h2 - h1_ref[...]) * 0.5
    mu = jnp.mean(d, axis=1, keepdims=True)
    dc = d - mu
    var = jnp.mean(dc * dc, axis=1, keepdims=True)
    z = dc / jnp.sqrt(var) * lng_ref[...][None, :] + lnb_ref[...][None, :]
    xr = h2 + jnp.dot(z, wavg_ref[...], preferred_element_type=jnp.float32) \
        + bavg_ref[...][None, :]
    mp3_ref[...] = sdis * jnp.dot(xr, w3_ref[...],
                                  preferred_element_type=jnp.float32)


_b3_call = pl.pallas_call(
    _b3_body,
    grid=(N // R,),
    in_specs=[
        pl.BlockSpec((R, 1), lambda i: (i, 0)),
        pl.BlockSpec((NC, R, 128), lambda i: (0, i, 0)),
        pl.BlockSpec((R, 128), lambda i: (i, 0)),
        pl.BlockSpec((R, 128), lambda i: (i, 0)),
        pl.BlockSpec((128,), lambda i: (0,)),
        pl.BlockSpec((128,), lambda i: (0,)),
        pl.BlockSpec((128,), lambda i: (0,)),
        pl.BlockSpec((128,), lambda i: (0,)),
        pl.BlockSpec((128, 128), lambda i: (0, 0)),
        pl.BlockSpec((128,), lambda i: (0,)),
        pl.BlockSpec((128, 128), lambda i: (0, 0)),
    ],
    out_specs=pl.BlockSpec((R, 128), lambda i: (i, 0)),
    out_shape=jax.ShapeDtypeStruct((N, 128), jnp.float32),
)


def _b4_body(dis_ref, acc_ref, mp3_ref, b3_ref, out_ref):
    sdis = dis_ref[...]
    out_ref[...] = sdis * (acc_ref[0][:, :64] + acc_ref[1][:, :64]
                           + mp3_ref[...][:, :64]) + b3_ref[...][None, :]


_b4_call = pl.pallas_call(
    _b4_body,
    grid=(N // R,),
    in_specs=[
        pl.BlockSpec((R, 1), lambda i: (i, 0)),
        pl.BlockSpec((NC, R, 128), lambda i: (0, i, 0)),
        pl.BlockSpec((R, 128), lambda i: (i, 0)),
        pl.BlockSpec((64,), lambda i: (0,)),
    ],
    out_specs=pl.BlockSpec((R, 64), lambda i: (i, 0)),
    out_shape=jax.ShapeDtypeStruct((N, 64), jnp.float32),
)


def kernel(x, adj, W1, b1, g1, be1, W2, b2, g2, be2,
           lng, lnb, Wavg, bavg, W3, b3):
    src = adj[0]
    dst = adj[1]
    npad = EPT - E
    ar = jnp.arange(npad, dtype=jnp.int32)
    srcp = jnp.concatenate([src, ar % N]).reshape(NW, NT, NB, CH)
    dstp_flat = jnp.concatenate([dst, N + ar % (NP - N)])
    dstp = dstp_flat.reshape(NW, NT, NB, CH)

    ard = jnp.arange(EPD - E, dtype=jnp.int32)
    dstp_deg = jnp.concatenate([dst, N + ard % (NP - N)]).reshape(
        NW, NCHD, 128)
    degparts = _deg_call(dstp_deg).reshape(NW, NP).T

    a1 = BNS * g1
    b1e = b1 * a1 + be1
    a2 = BNS * g2
    b2e = b2 * a2 + be2

    mp1, dis = _b1_call(degparts, x, W1)
    acc1 = _scat128(mp1, srcp, dstp)
    h1, mp2 = _b2_call(dis, acc1, mp1, a1, b1e, W2)
    acc2 = _scat128(mp2, srcp, dstp)
    W3p = jnp.pad(W3, ((0, 0), (0, 64)))
    mp3 = _b3_call(dis, acc2, mp2, h1, a2, b2e, lng, lnb, Wavg, bavg, W3p)
    acc3 = _scat128(mp3, srcp, dstp)
    out = _b4_call(dis, acc3, mp3, b3)
    return out


# speedup vs baseline: 1.0389x; 1.0256x over previous
import functools

import jax
import jax.numpy as jnp
import numpy as np
from jax import lax
from jax.experimental import pallas as pl
from jax.experimental.pallas import tpu as pltpu
from jax.experimental.pallas import tpu_sc as plsc

N = 10000
E = 320000
NP = 10016
NC = 2
NS = 16
NW = NC * NS
CH = 120
NB = 3
NCH = 84
NT = NCH // NB
EPT = NW * NCH * CH
NCHD = 79
EPD = NW * NCHD * 128
NPD = 10112
BNS = 1.0 / (1.0 + 1e-5) ** 0.5
R = 1000

_mesh = plsc.VectorSubcoreMesh(core_axis_name="c", subcore_axis_name="s")
_sc_params = pltpu.CompilerParams(needs_layout_passes=False)

_PAD_SRC = np.arange(EPT - E, dtype=np.int32) % N
_PAD_DST = (N + np.arange(EPT - E, dtype=np.int32) % (NP - N)).astype(np.int32)
_PAD_DSTD = (N + np.arange(EPD - E, dtype=np.int32) % (NP - N)).astype(
    np.int32)


def _deg_body(dst_hbm, deg_hbm, dstbuf, degbuf, stage, colbuf, sumbuf):
    c = lax.axis_index("c")
    s = lax.axis_index("s")
    wid = c * NS + s
    pltpu.sync_copy(dst_hbm.at[wid], dstbuf)
    z = jnp.zeros((16,), jnp.float32)

    def zero(i, carry):
        degbuf[pl.ds(i * 16, 16)] = z
        return carry

    lax.fori_loop(0, NPD // 16, zero, 0)
    ones = jnp.ones((16,), jnp.float32)

    def body(g, carry):
        for k in range(8):
            idx = dstbuf[g, pl.ds(k * 16, 16)]
            plsc.addupdate_scatter(degbuf, [idx], ones)
        return carry

    lax.fori_loop(0, NCHD, body, 0)
    pltpu.sync_copy(degbuf, stage.at[s])
    plsc.subcore_barrier()
    cw = 640

    @pl.when(s == 15)
    def _():
        pltpu.sync_copy(stage.at[:, pl.ds(15 * cw, NPD - 15 * cw)],
                        colbuf.at[:, pl.ds(0, NPD - 15 * cw)])

    @pl.when(s < 15)
    def _():
        pltpu.sync_copy(stage.at[:, pl.ds(s * cw, cw)], colbuf)

    ngrp = jnp.where(s < 15, cw // 16, (NPD - 15 * cw) // 16)

    def red(g, carry):
        v = colbuf[0, pl.ds(g * 16, 16)]
        for k in range(1, NS):
            v = v + colbuf[k, pl.ds(g * 16, 16)]
        sumbuf[pl.ds(g * 16, 16)] = v
        return carry

    lax.fori_loop(0, ngrp, red, 0)

    @pl.when(s < 15)
    def _():
        pltpu.sync_copy(sumbuf, deg_hbm.at[c, pl.ds(s * cw, cw)])

    @pl.when(s == 15)
    def _():
        pltpu.sync_copy(sumbuf.at[pl.ds(0, NPD - 15 * cw)],
                        deg_hbm.at[c, pl.ds(15 * cw, NPD - 15 * cw)])


_deg_call = pl.kernel(
    _deg_body,
    out_type=jax.ShapeDtypeStruct((NC, NPD), jnp.float32),
    mesh=_mesh,
    compiler_params=_sc_params,
    scratch_types=[
        pltpu.VMEM((NCHD, 128), jnp.int32),
        pltpu.VMEM((NPD,), jnp.float32),
        pltpu.VMEM_SHARED((NS, NPD), jnp.float32),
        pltpu.VMEM((NS, 640), jnp.float32),
        pltpu.VMEM((640,), jnp.float32),
    ],
)


def _make_scatter(F):
    def body(mp_hbm, src_hbm, dst_hbm, out_hbm, idxbuf, *rest):
        rows = rest[:NB]
        acc = rest[NB]
        gsems = rest[NB + 1:2 * NB + 1]
        isem = rest[2 * NB + 1]
        rows0 = rows[0]
        c = lax.axis_index("c")
        s = lax.axis_index("s")
        wid = c * NS + s

        z = jnp.zeros((16,), jnp.float32)

        def zero(i, carry):
            for k in range(F // 16):
                rows0[i, pl.ds(k * 16, 16)] = z
            return carry

        lax.fori_loop(0, CH, zero, 0)
        base = s * 632

        def _zero_slice(nrows):
            nf = nrows // CH
            rm = nrows - nf * CH
            for k in range(nf):
                pltpu.sync_copy(rows0, acc.at[pl.ds(base + k * CH, CH)])
            if rm:
                pltpu.sync_copy(rows0.at[pl.ds(0, rm)],
                                acc.at[pl.ds(base + nf * CH, rm)])

        @pl.when(s < 15)
        def _():
            _zero_slice(632)

        @pl.when(s == 15)
        def _():
            _zero_slice(536)

        plsc.subcore_barrier()

        pltpu.sync_copy(src_hbm.at[wid, 0], idxbuf.at[pl.ds(0, NB)])
        pltpu.sync_copy(dst_hbm.at[wid, 0], idxbuf.at[pl.ds(2 * NB, NB)])
        for j in range(NB - 1):
            pltpu.async_copy(mp_hbm.at[idxbuf.at[j]], rows[j], gsems[j])

        def macro(t, carry):
            slot = lax.rem(t, 2)
            nslot = 1 - slot
            nxt = t + 1

            @pl.when(nxt < NT)
            def _():
                pltpu.async_copy(src_hbm.at[wid, nxt],
                                 idxbuf.at[pl.ds(nslot * NB, NB)], isem)
                pltpu.async_copy(dst_hbm.at[wid, nxt],
                                 idxbuf.at[pl.ds(2 * NB + nslot * NB, NB)],
                                 isem)

            for j in range(NB):
                pltpu.make_async_copy(mp_hbm.at[idxbuf.at[slot * NB + j]],
                                      rows[j], gsems[j]).wait()
                if j == 0:
                    pltpu.async_copy(mp_hbm.at[idxbuf.at[slot * NB + NB - 1]],
                                     rows[NB - 1], gsems[NB - 1])
                elif j == 1:
                    @pl.when(nxt < NT)
                    def _():
                        pltpu.make_async_copy(
                            src_hbm.at[wid, nxt],
                            idxbuf.at[pl.ds(nslot * NB, NB)], isem).wait()
                        pltpu.make_async_copy(
                            dst_hbm.at[wid, nxt],
                            idxbuf.at[pl.ds(2 * NB + nslot * NB, NB)],
                            isem).wait()
                        pltpu.async_copy(mp_hbm.at[idxbuf.at[nslot * NB]],
                                         rows[0], gsems[0])
                else:
                    @pl.when(nxt < NT)
                    def _():
                        pltpu.async_copy(
                            mp_hbm.at[idxbuf.at[nslot * NB + j - 1]],
                            rows[j - 1], gsems[j - 1])
                pltpu.sync_copy(rows[j],
                                acc.at[idxbuf.at[2 * NB + slot * NB + j]],
                                add=True)
            return carry

        lax.fori_loop(0, NT, macro, 0)
        plsc.subcore_barrier()

        @pl.when(s < 15)
        def _():
            pltpu.sync_copy(acc.at[pl.ds(base, 632)],
                            out_hbm.at[c, pl.ds(base, 632)])

        @pl.when(s == 15)
        def _():
            pltpu.sync_copy(acc.at[pl.ds(base, 536)],
                            out_hbm.at[c, pl.ds(base, 536)])

    return pl.kernel(
        body,
        out_type=jax.ShapeDtypeStruct((NC, NP, F), jnp.float32),
        mesh=_mesh,
        compiler_params=_sc_params,
        scratch_types=(
            [pltpu.VMEM((4 * NB, CH), jnp.int32)]
            + [pltpu.VMEM((CH, F), jnp.float32) for _ in range(NB)]
            + [pltpu.VMEM_SHARED((NP, F), jnp.float32)]
            + [pltpu.SemaphoreType.DMA for _ in range(NB + 1)]
        ),
    )


_scat128 = _make_scatter(128)


def _b1_body(deg_ref, x_ref, w_ref, mp_ref, dis_ref):
    dis = lax.rsqrt(1.0 + deg_ref[...])
    dis_ref[...] = dis
    m = jnp.dot(x_ref[...], w_ref[...], preferred_element_type=jnp.float32)
    mp_ref[...] = m * dis


_b1_call = pl.pallas_call(
    _b1_body,
    grid=(N // R,),
    in_specs=[
        pl.BlockSpec((R, 1), lambda i: (i, 0)),
        pl.BlockSpec((R, 128), lambda i: (i, 0)),
        pl.BlockSpec((128, 128), lambda i: (0, 0)),
    ],
    out_specs=[
        pl.BlockSpec((R, 128), lambda i: (i, 0)),
        pl.BlockSpec((R, 1), lambda i: (i, 0)),
    ],
    out_shape=[
        jax.ShapeDtypeStruct((N, 128), jnp.float32),
        jax.ShapeDtypeStruct((N, 1), jnp.float32),
    ],
)


def _b2_body(dis_ref, acc_ref, mp1_ref, a1_ref, b1_ref, w2_ref,
             h1_ref, mp2_ref):
    sdis = dis_ref[...]
    conv = sdis * (acc_ref[0] + acc_ref[1] + mp1_ref[...])
    h1 = jnp.maximum(conv * a1_ref[...][None, :] + b1_ref[...][None, :], 0.0)
    h1_ref[...] = h1
    mp2_ref[...] = sdis * jnp.dot(h1, w2_ref[...],
                                  preferred_element_type=jnp.float32)


_b2_call = pl.pallas_call(
    _b2_body,
    grid=(N // R,),
    in_specs=[
        pl.BlockSpec((R, 1), lambda i: (i, 0)),
        pl.BlockSpec((NC, R, 128), lambda i: (0, i, 0)),
        pl.BlockSpec((R, 128), lambda i: (i, 0)),
        pl.BlockSpec((128,), lambda i: (0,)),
        pl.BlockSpec((128,), lambda i: (0,)),
        pl.BlockSpec((128, 128), lambda i: (0, 0)),
    ],
    out_specs=[
        pl.BlockSpec((R, 128), lambda i: (i, 0)),
        pl.BlockSpec((R, 128), lambda i: (i, 0)),
    ],
    out_shape=[
        jax.ShapeDtypeStruct((N, 128), jnp.float32),
        jax.ShapeDtypeStruct((N, 128), jnp.float32),
    ],
)


def _b3_body(dis_ref, acc_ref, mp2_ref, h1_ref, a2_ref, b2_ref,
             lng_ref, lnb_ref, wavg_ref, bavg_ref, w3_ref, mp3_ref):
    sdis = dis_ref[...]
    conv = sdis * (acc_ref[0] + acc_ref[1] + mp2_ref[...])
    h2 = jnp.maximum(conv * a2_ref[...][None, :] + b2_ref[...][None, :], 0.0)
    d = (h2 - h1_ref[...]) * 0.5
    mu = jnp.mean(d, axis=1, keepdims=True)
    dc = d - mu
    var = jnp.mean(dc * dc, axis=1, keepdims=True)
    z = dc / jnp.sqrt(var) * lng_ref[...][None, :] + lnb_ref[...][None, :]
    xr = h2 + jnp.dot(z, wavg_ref[...], preferred_element_type=jnp.float32) \
        + bavg_ref[...][None, :]
    mp3_ref[...] = sdis * jnp.dot(xr, w3_ref[...],
                                  preferred_element_type=jnp.float32)


_b3_call = pl.pallas_call(
    _b3_body,
    grid=(N // R,),
    in_specs=[
        pl.BlockSpec((R, 1), lambda i: (i, 0)),
        pl.BlockSpec((NC, R, 128), lambda i: (0, i, 0)),
        pl.BlockSpec((R, 128), lambda i: (i, 0)),
        pl.BlockSpec((R, 128), lambda i: (i, 0)),
        pl.BlockSpec((128,), lambda i: (0,)),
        pl.BlockSpec((128,), lambda i: (0,)),
        pl.BlockSpec((128,), lambda i: (0,)),
        pl.BlockSpec((128,), lambda i: (0,)),
        pl.BlockSpec((128, 128), lambda i: (0, 0)),
        pl.BlockSpec((128,), lambda i: (0,)),
        pl.BlockSpec((128, 128), lambda i: (0, 0)),
    ],
    out_specs=pl.BlockSpec((R, 128), lambda i: (i, 0)),
    out_shape=jax.ShapeDtypeStruct((N, 128), jnp.float32),
)


def _b4_body(dis_ref, acc_ref, mp3_ref, b3_ref, out_ref):
    sdis = dis_ref[...]
    out_ref[...] = sdis * (acc_ref[0][:, :64] + acc_ref[1][:, :64]
                           + mp3_ref[...][:, :64]) + b3_ref[...][None, :]


_b4_call = pl.pallas_call(
    _b4_body,
    grid=(N // R,),
    in_specs=[
        pl.BlockSpec((R, 1), lambda i: (i, 0)),
        pl.BlockSpec((NC, R, 128), lambda i: (0, i, 0)),
        pl.BlockSpec((R, 128), lambda i: (i, 0)),
        pl.BlockSpec((64,), lambda i: (0,)),
    ],
    out_specs=pl.BlockSpec((R, 64), lambda i: (i, 0)),
    out_shape=jax.ShapeDtypeStruct((N, 64), jnp.float32),
)


def kernel(x, adj, W1, b1, g1, be1, W2, b2, g2, be2,
           lng, lnb, Wavg, bavg, W3, b3):
    src = adj[0]
    dst = adj[1]
    srcp = jnp.concatenate([src, _PAD_SRC]).reshape(NW, NT, NB, CH)
    dstp = jnp.concatenate([dst, _PAD_DST]).reshape(NW, NT, NB, CH)
    dstp_deg = jnp.concatenate([dst, _PAD_DSTD]).reshape(NW, NCHD, 128)
    degparts = _deg_call(dstp_deg)
    degsum = (degparts[0] + degparts[1]).reshape(NPD, 1)

    a1 = BNS * g1
    b1e = b1 * a1 + be1
    a2 = BNS * g2
    b2e = b2 * a2 + be2

    mp1, dis = _b1_call(degsum, x, W1)
    acc1 = _scat128(mp1, srcp, dstp)
    h1, mp2 = _b2_call(dis, acc1, mp1, a1, b1e, W2)
    acc2 = _scat128(mp2, srcp, dstp)
    W3p = jnp.pad(W3, ((0, 0), (0, 64)))
    mp3 = _b3_call(dis, acc2, mp2, h1, a2, b2e, lng, lnb, Wavg, bavg, W3p)
    acc3 = _scat128(mp3, srcp, dstp)
    out = _b4_call(dis, acc3, mp3, b3)
    return out


# speedup vs baseline: 1.0685x; 1.0286x over previous
import functools

import jax
import jax.numpy as jnp
import numpy as np
from jax import lax
from jax.experimental import pallas as pl
from jax.experimental.pallas import tpu as pltpu
from jax.experimental.pallas import tpu_sc as plsc

N = 10000
E = 320000
NP = 10016
NC = 2
NS = 16
NW = NC * NS
CH = 120
NB = 3
NCH = 84
NT = NCH // NB
EPT = NW * NCH * CH
NCHD = 79
EPD = NW * NCHD * 128
NPD = 10112
BNS = 1.0 / (1.0 + 1e-5) ** 0.5
R = 2000

_mesh = plsc.VectorSubcoreMesh(core_axis_name="c", subcore_axis_name="s")
_sc_params = pltpu.CompilerParams(needs_layout_passes=False)

_PAD_SRC = np.arange(EPT - E, dtype=np.int32) % N
_PAD_DST = (N + np.arange(EPT - E, dtype=np.int32) % (NP - N)).astype(np.int32)
_PAD_DSTD = (N + np.arange(EPD - E, dtype=np.int32) % (NP - N)).astype(
    np.int32)


def _deg_body(dst_hbm, deg_hbm, dstbuf, degbuf, stage, colbuf, sumbuf):
    c = lax.axis_index("c")
    s = lax.axis_index("s")
    wid = c * NS + s
    pltpu.sync_copy(dst_hbm.at[wid], dstbuf)
    z = jnp.zeros((16,), jnp.float32)

    def zero(i, carry):
        degbuf[pl.ds(i * 16, 16)] = z
        return carry

    lax.fori_loop(0, NPD // 16, zero, 0)
    ones = jnp.ones((16,), jnp.float32)

    def body(g, carry):
        for k in range(8):
            idx = dstbuf[g, pl.ds(k * 16, 16)]
            plsc.addupdate_scatter(degbuf, [idx], ones)
        return carry

    lax.fori_loop(0, NCHD, body, 0)
    pltpu.sync_copy(degbuf, stage.at[s])
    plsc.subcore_barrier()
    cw = 640

    @pl.when(s == 15)
    def _():
        pltpu.sync_copy(stage.at[:, pl.ds(15 * cw, NPD - 15 * cw)],
                        colbuf.at[:, pl.ds(0, NPD - 15 * cw)])

    @pl.when(s < 15)
    def _():
        pltpu.sync_copy(stage.at[:, pl.ds(s * cw, cw)], colbuf)

    ngrp = jnp.where(s < 15, cw // 16, (NPD - 15 * cw) // 16)

    def red(g, carry):
        v = colbuf[0, pl.ds(g * 16, 16)]
        for k in range(1, NS):
            v = v + colbuf[k, pl.ds(g * 16, 16)]
        sumbuf[pl.ds(g * 16, 16)] = v
        return carry

    lax.fori_loop(0, ngrp, red, 0)

    @pl.when(s < 15)
    def _():
        pltpu.sync_copy(sumbuf, deg_hbm.at[c, pl.ds(s * cw, cw)])

    @pl.when(s == 15)
    def _():
        pltpu.sync_copy(sumbuf.at[pl.ds(0, NPD - 15 * cw)],
                        deg_hbm.at[c, pl.ds(15 * cw, NPD - 15 * cw)])


_deg_call = pl.kernel(
    _deg_body,
    out_type=jax.ShapeDtypeStruct((NC, NPD), jnp.float32),
    mesh=_mesh,
    compiler_params=_sc_params,
    scratch_types=[
        pltpu.VMEM((NCHD, 128), jnp.int32),
        pltpu.VMEM((NPD,), jnp.float32),
        pltpu.VMEM_SHARED((NS, NPD), jnp.float32),
        pltpu.VMEM((NS, 640), jnp.float32),
        pltpu.VMEM((640,), jnp.float32),
    ],
)


def _make_scatter(F):
    def body(mp_hbm, src_hbm, dst_hbm, out_hbm, idxbuf, *rest):
        rows = rest[:NB]
        acc = rest[NB]
        gsems = rest[NB + 1:2 * NB + 1]
        isem = rest[2 * NB + 1]
        rows0 = rows[0]
        c = lax.axis_index("c")
        s = lax.axis_index("s")
        wid = c * NS + s

        z = jnp.zeros((16,), jnp.float32)

        def zero(i, carry):
            for k in range(F // 16):
                rows0[i, pl.ds(k * 16, 16)] = z
            return carry

        lax.fori_loop(0, CH, zero, 0)
        base = s * 632

        def _zero_slice(nrows):
            nf = nrows // CH
            rm = nrows - nf * CH
            for k in range(nf):
                pltpu.sync_copy(rows0, acc.at[pl.ds(base + k * CH, CH)])
            if rm:
                pltpu.sync_copy(rows0.at[pl.ds(0, rm)],
                                acc.at[pl.ds(base + nf * CH, rm)])

        @pl.when(s < 15)
        def _():
            _zero_slice(632)

        @pl.when(s == 15)
        def _():
            _zero_slice(536)

        plsc.subcore_barrier()

        pltpu.sync_copy(src_hbm.at[wid, 0], idxbuf.at[pl.ds(0, NB)])
        pltpu.sync_copy(dst_hbm.at[wid, 0], idxbuf.at[pl.ds(2 * NB, NB)])
        for j in range(NB - 1):
            pltpu.async_copy(mp_hbm.at[idxbuf.at[j]], rows[j], gsems[j])

        def macro(t, carry):
            slot = lax.rem(t, 2)
            nslot = 1 - slot
            nxt = t + 1

            @pl.when(nxt < NT)
            def _():
                pltpu.async_copy(src_hbm.at[wid, nxt],
                                 idxbuf.at[pl.ds(nslot * NB, NB)], isem)
                pltpu.async_copy(dst_hbm.at[wid, nxt],
                                 idxbuf.at[pl.ds(2 * NB + nslot * NB, NB)],
                                 isem)

            for j in range(NB):
                pltpu.make_async_copy(mp_hbm.at[idxbuf.at[slot * NB + j]],
                                      rows[j], gsems[j]).wait()
                if j == 0:
                    pltpu.async_copy(mp_hbm.at[idxbuf.at[slot * NB + NB - 1]],
                                     rows[NB - 1], gsems[NB - 1])
                elif j == 1:
                    @pl.when(nxt < NT)
                    def _():
                        pltpu.make_async_copy(
                            src_hbm.at[wid, nxt],
                            idxbuf.at[pl.ds(nslot * NB, NB)], isem).wait()
                        pltpu.make_async_copy(
                            dst_hbm.at[wid, nxt],
                            idxbuf.at[pl.ds(2 * NB + nslot * NB, NB)],
                            isem).wait()
                        pltpu.async_copy(mp_hbm.at[idxbuf.at[nslot * NB]],
                                         rows[0], gsems[0])
                else:
                    @pl.when(nxt < NT)
                    def _():
                        pltpu.async_copy(
                            mp_hbm.at[idxbuf.at[nslot * NB + j - 1]],
                            rows[j - 1], gsems[j - 1])
                pltpu.sync_copy(rows[j],
                                acc.at[idxbuf.at[2 * NB + slot * NB + j]],
                                add=True)
            return carry

        lax.fori_loop(0, NT, macro, 0)
        plsc.subcore_barrier()

        @pl.when(s < 15)
        def _():
            pltpu.sync_copy(acc.at[pl.ds(base, 632)],
                            out_hbm.at[c, pl.ds(base, 632)])

        @pl.when(s == 15)
        def _():
            pltpu.sync_copy(acc.at[pl.ds(base, 536)],
                            out_hbm.at[c, pl.ds(base, 536)])

    return pl.kernel(
        body,
        out_type=jax.ShapeDtypeStruct((NC, NP, F), jnp.float32),
        mesh=_mesh,
        compiler_params=_sc_params,
        scratch_types=(
            [pltpu.VMEM((4 * NB, CH), jnp.int32)]
            + [pltpu.VMEM((CH, F), jnp.float32) for _ in range(NB)]
            + [pltpu.VMEM_SHARED((NP, F), jnp.float32)]
            + [pltpu.SemaphoreType.DMA for _ in range(NB + 1)]
        ),
    )


_scat128 = _make_scatter(128)


def _b1_body(deg_ref, x_ref, w_ref, mp_ref, dis_ref):
    dis = lax.rsqrt(1.0 + deg_ref[...])
    dis_ref[...] = dis
    m = jnp.dot(x_ref[...], w_ref[...], preferred_element_type=jnp.float32)
    mp_ref[...] = m * dis


_b1_call = pl.pallas_call(
    _b1_body,
    grid=(N // R,),
    in_specs=[
        pl.BlockSpec((R, 1), lambda i: (i, 0)),
        pl.BlockSpec((R, 128), lambda i: (i, 0)),
        pl.BlockSpec((128, 128), lambda i: (0, 0)),
    ],
    out_specs=[
        pl.BlockSpec((R, 128), lambda i: (i, 0)),
        pl.BlockSpec((R, 1), lambda i: (i, 0)),
    ],
    out_shape=[
        jax.ShapeDtypeStruct((N, 128), jnp.float32),
        jax.ShapeDtypeStruct((N, 1), jnp.float32),
    ],
)


def _b2_body(dis_ref, acc_ref, mp1_ref, a1_ref, b1_ref, w2_ref,
             h1_ref, mp2_ref):
    sdis = dis_ref[...]
    conv = sdis * (acc_ref[0] + acc_ref[1] + mp1_ref[...])
    h1 = jnp.maximum(conv * a1_ref[...][None, :] + b1_ref[...][None, :], 0.0)
    h1_ref[...] = h1
    mp2_ref[...] = sdis * jnp.dot(h1, w2_ref[...],
                                  preferred_element_type=jnp.float32)


_b2_call = pl.pallas_call(
    _b2_body,
    grid=(N // R,),
    in_specs=[
        pl.BlockSpec((R, 1), lambda i: (i, 0)),
        pl.BlockSpec((NC, R, 128), lambda i: (0, i, 0)),
        pl.BlockSpec((R, 128), lambda i: (i, 0)),
        pl.BlockSpec((128,), lambda i: (0,)),
        pl.BlockSpec((128,), lambda i: (0,)),
        pl.BlockSpec((128, 128), lambda i: (0, 0)),
    ],
    out_specs=[
        pl.BlockSpec((R, 128), lambda i: (i, 0)),
        pl.BlockSpec((R, 128), lambda i: (i, 0)),
    ],
    out_shape=[
        jax.ShapeDtypeStruct((N, 128), jnp.float32),
        jax.ShapeDtypeStruct((N, 128), jnp.float32),
    ],
)


def _b3_body(dis_ref, acc_ref, mp2_ref, h1_ref, a2_ref, b2_ref,
             lng_ref, lnb_ref, wavg_ref, bavg_ref, w3_ref, mp3_ref):
    sdis = dis_ref[...]
    conv = sdis * (acc_ref[0] + acc_ref[1] + mp2_ref[...])
    h2 = jnp.maximum(conv * a2_ref[...][None, :] + b2_ref[...][None, :], 0.0)
    d = (h2 - h1_ref[...]) * 0.5
    mu = jnp.mean(d, axis=1, keepdims=True)
    dc = d - mu
    var = jnp.mean(dc * dc, axis=1, keepdims=True)
    z = dc / jnp.sqrt(var) * lng_ref[...][None, :] + lnb_ref[...][None, :]
    xr = h2 + jnp.dot(z, wavg_ref[...], preferred_element_type=jnp.float32) \
        + bavg_ref[...][None, :]
    mp3_ref[...] = sdis * jnp.dot(xr, w3_ref[...],
                                  preferred_element_type=jnp.float32)


_b3_call = pl.pallas_call(
    _b3_body,
    grid=(N // R,),
    in_specs=[
        pl.BlockSpec((R, 1), lambda i: (i, 0)),
        pl.BlockSpec((NC, R, 128), lambda i: (0, i, 0)),
        pl.BlockSpec((R, 128), lambda i: (i, 0)),
        pl.BlockSpec((R, 128), lambda i: (i, 0)),
        pl.BlockSpec((128,), lambda i: (0,)),
        pl.BlockSpec((128,), lambda i: (0,)),
        pl.BlockSpec((128,), lambda i: (0,)),
        pl.BlockSpec((128,), lambda i: (0,)),
        pl.BlockSpec((128, 128), lambda i: (0, 0)),
        pl.BlockSpec((128,), lambda i: (0,)),
        pl.BlockSpec((128, 128), lambda i: (0, 0)),
    ],
    out_specs=pl.BlockSpec((R, 128), lambda i: (i, 0)),
    out_shape=jax.ShapeDtypeStruct((N, 128), jnp.float32),
)


def _b4_body(dis_ref, acc_ref, mp3_ref, b3_ref, out_ref):
    sdis = dis_ref[...]
    out_ref[...] = sdis * (acc_ref[0][:, :64] + acc_ref[1][:, :64]
                           + mp3_ref[...][:, :64]) + b3_ref[...][None, :]


_b4_call = pl.pallas_call(
    _b4_body,
    grid=(N // R,),
    in_specs=[
        pl.BlockSpec((R, 1), lambda i: (i, 0)),
        pl.BlockSpec((NC, R, 128), lambda i: (0, i, 0)),
        pl.BlockSpec((R, 128), lambda i: (i, 0)),
        pl.BlockSpec((64,), lambda i: (0,)),
    ],
    out_specs=pl.BlockSpec((R, 64), lambda i: (i, 0)),
    out_shape=jax.ShapeDtypeStruct((N, 64), jnp.float32),
)


def kernel(x, adj, W1, b1, g1, be1, W2, b2, g2, be2,
           lng, lnb, Wavg, bavg, W3, b3):
    src = adj[0]
    dst = adj[1]
    srcp = jnp.concatenate([src, _PAD_SRC]).reshape(NW, NT, NB, CH)
    dstp = jnp.concatenate([dst, _PAD_DST]).reshape(NW, NT, NB, CH)
    dstp_deg = jnp.concatenate([dst, _PAD_DSTD]).reshape(NW, NCHD, 128)
    degparts = _deg_call(dstp_deg)
    degsum = (degparts[0] + degparts[1]).reshape(NPD, 1)

    a1 = BNS * g1
    b1e = b1 * a1 + be1
    a2 = BNS * g2
    b2e = b2 * a2 + be2

    mp1, dis = _b1_call(degsum, x, W1)
    acc1 = _scat128(mp1, srcp, dstp)
    h1, mp2 = _b2_call(dis, acc1, mp1, a1, b1e, W2)
    acc2 = _scat128(mp2, srcp, dstp)
    W3p = jnp.pad(W3, ((0, 0), (0, 64)))
    mp3 = _b3_call(dis, acc2, mp2, h1, a2, b2e, lng, lnb, Wavg, bavg, W3p)
    acc3 = _scat128(mp3, srcp, dstp)
    out = _b4_call(dis, acc3, mp3, b3)
    return out


# speedup vs baseline: 1.0806x; 1.0113x over previous
import functools

import jax
import jax.numpy as jnp
import numpy as np
from jax import lax
from jax.experimental import pallas as pl
from jax.experimental.pallas import tpu as pltpu
from jax.experimental.pallas import tpu_sc as plsc

N = 10000
E = 320000
NP = 10016
NC = 2
NS = 16
NW = NC * NS
CH = 120
NB = 3
NCH = 84
NT = NCH // NB
EPT = NW * NCH * CH
NCHD = 79
EPD = NW * NCHD * 128
NPD = 10112
BNS = 1.0 / (1.0 + 1e-5) ** 0.5
R = 5000

_mesh = plsc.VectorSubcoreMesh(core_axis_name="c", subcore_axis_name="s")
_sc_params = pltpu.CompilerParams(needs_layout_passes=False)

_PAD_SRC = np.arange(EPT - E, dtype=np.int32) % N
_PAD_DST = (N + np.arange(EPT - E, dtype=np.int32) % (NP - N)).astype(np.int32)
_PAD_DSTD = (N + np.arange(EPD - E, dtype=np.int32) % (NP - N)).astype(
    np.int32)


def _deg_body(dst_hbm, deg_hbm, dstbuf, degbuf, stage, colbuf, sumbuf):
    c = lax.axis_index("c")
    s = lax.axis_index("s")
    wid = c * NS + s
    pltpu.sync_copy(dst_hbm.at[wid], dstbuf)
    z = jnp.zeros((16,), jnp.float32)

    def zero(i, carry):
        degbuf[pl.ds(i * 16, 16)] = z
        return carry

    lax.fori_loop(0, NPD // 16, zero, 0)
    ones = jnp.ones((16,), jnp.float32)

    def body(g, carry):
        for k in range(8):
            idx = dstbuf[g, pl.ds(k * 16, 16)]
            plsc.addupdate_scatter(degbuf, [idx], ones)
        return carry

    lax.fori_loop(0, NCHD, body, 0)
    pltpu.sync_copy(degbuf, stage.at[s])
    plsc.subcore_barrier()
    cw = 640

    @pl.when(s == 15)
    def _():
        pltpu.sync_copy(stage.at[:, pl.ds(15 * cw, NPD - 15 * cw)],
                        colbuf.at[:, pl.ds(0, NPD - 15 * cw)])

    @pl.when(s < 15)
    def _():
        pltpu.sync_copy(stage.at[:, pl.ds(s * cw, cw)], colbuf)

    ngrp = jnp.where(s < 15, cw // 16, (NPD - 15 * cw) // 16)

    def red(g, carry):
        v = colbuf[0, pl.ds(g * 16, 16)]
        for k in range(1, NS):
            v = v + colbuf[k, pl.ds(g * 16, 16)]
        sumbuf[pl.ds(g * 16, 16)] = v
        return carry

    lax.fori_loop(0, ngrp, red, 0)

    @pl.when(s < 15)
    def _():
        pltpu.sync_copy(sumbuf, deg_hbm.at[c, pl.ds(s * cw, cw)])

    @pl.when(s == 15)
    def _():
        pltpu.sync_copy(sumbuf.at[pl.ds(0, NPD - 15 * cw)],
                        deg_hbm.at[c, pl.ds(15 * cw, NPD - 15 * cw)])


_deg_call = pl.kernel(
    _deg_body,
    out_type=jax.ShapeDtypeStruct((NC, NPD), jnp.float32),
    mesh=_mesh,
    compiler_params=_sc_params,
    scratch_types=[
        pltpu.VMEM((NCHD, 128), jnp.int32),
        pltpu.VMEM((NPD,), jnp.float32),
        pltpu.VMEM_SHARED((NS, NPD), jnp.float32),
        pltpu.VMEM((NS, 640), jnp.float32),
        pltpu.VMEM((640,), jnp.float32),
    ],
)


def _make_scatter(F):
    def body(mp_hbm, src_hbm, dst_hbm, out_hbm, idxbuf, *rest):
        rows = rest[:NB]
        acc = rest[NB]
        gsems = rest[NB + 1:2 * NB + 1]
        isem = rest[2 * NB + 1]
        rows0 = rows[0]
        c = lax.axis_index("c")
        s = lax.axis_index("s")
        wid = c * NS + s

        z = jnp.zeros((16,), jnp.float32)

        def zero(i, carry):
            for k in range(F // 16):
                rows0[i, pl.ds(k * 16, 16)] = z
            return carry

        lax.fori_loop(0, CH, zero, 0)
        base = s * 632

        def _zero_slice(nrows):
            nf = nrows // CH
            rm = nrows - nf * CH
            for k in range(nf):
                pltpu.sync_copy(rows0, acc.at[pl.ds(base + k * CH, CH)])
            if rm:
                pltpu.sync_copy(rows0.at[pl.ds(0, rm)],
                                acc.at[pl.ds(base + nf * CH, rm)])

        @pl.when(s < 15)
        def _():
            _zero_slice(632)

        @pl.when(s == 15)
        def _():
            _zero_slice(536)

        plsc.subcore_barrier()

        pltpu.sync_copy(src_hbm.at[wid, 0], idxbuf.at[pl.ds(0, NB)])
        pltpu.sync_copy(dst_hbm.at[wid, 0], idxbuf.at[pl.ds(2 * NB, NB)])
        for j in range(NB - 1):
            pltpu.async_copy(mp_hbm.at[idxbuf.at[j]], rows[j], gsems[j])

        def macro(t, carry):
            slot = lax.rem(t, 2)
            nslot = 1 - slot
            nxt = t + 1

            @pl.when(nxt < NT)
            def _():
                pltpu.async_copy(src_hbm.at[wid, nxt],
                                 idxbuf.at[pl.ds(nslot * NB, NB)], isem)
                pltpu.async_copy(dst_hbm.at[wid, nxt],
                                 idxbuf.at[pl.ds(2 * NB + nslot * NB, NB)],
                                 isem)

            for j in range(NB):
                pltpu.make_async_copy(mp_hbm.at[idxbuf.at[slot * NB + j]],
                                      rows[j], gsems[j]).wait()
                if j == 0:
                    pltpu.async_copy(mp_hbm.at[idxbuf.at[slot * NB + NB - 1]],
                                     rows[NB - 1], gsems[NB - 1])
                elif j == 1:
                    @pl.when(nxt < NT)
                    def _():
                        pltpu.make_async_copy(
                            src_hbm.at[wid, nxt],
                            idxbuf.at[pl.ds(nslot * NB, NB)], isem).wait()
                        pltpu.make_async_copy(
                            dst_hbm.at[wid, nxt],
                            idxbuf.at[pl.ds(2 * NB + nslot * NB, NB)],
                            isem).wait()
                        pltpu.async_copy(mp_hbm.at[idxbuf.at[nslot * NB]],
                                         rows[0], gsems[0])
                else:
                    @pl.when(nxt < NT)
                    def _():
                        pltpu.async_copy(
                            mp_hbm.at[idxbuf.at[nslot * NB + j - 1]],
                            rows[j - 1], gsems[j - 1])
                pltpu.sync_copy(rows[j],
                                acc.at[idxbuf.at[2 * NB + slot * NB + j]],
                                add=True)
            return carry

        lax.fori_loop(0, NT, macro, 0)
        plsc.subcore_barrier()

        @pl.when(s < 15)
        def _():
            pltpu.sync_copy(acc.at[pl.ds(base, 632)],
                            out_hbm.at[c, pl.ds(base, 632)])

        @pl.when(s == 15)
        def _():
            pltpu.sync_copy(acc.at[pl.ds(base, 536)],
                            out_hbm.at[c, pl.ds(base, 536)])

    return pl.kernel(
        body,
        out_type=jax.ShapeDtypeStruct((NC, NP, F), jnp.float32),
        mesh=_mesh,
        compiler_params=_sc_params,
        scratch_types=(
            [pltpu.VMEM((4 * NB, CH), jnp.int32)]
            + [pltpu.VMEM((CH, F), jnp.float32) for _ in range(NB)]
            + [pltpu.VMEM_SHARED((NP, F), jnp.float32)]
            + [pltpu.SemaphoreType.DMA for _ in range(NB + 1)]
        ),
    )


_scat128 = _make_scatter(128)


def _b1_body(deg_ref, x_ref, w_ref, mp_ref, dis_ref):
    dis = lax.rsqrt(1.0 + deg_ref[...])
    dis_ref[...] = dis
    m = jnp.dot(x_ref[...], w_ref[...], preferred_element_type=jnp.float32)
    mp_ref[...] = m * dis


_b1_call = pl.pallas_call(
    _b1_body,
    grid=(N // R,),
    in_specs=[
        pl.BlockSpec((R, 1), lambda i: (i, 0)),
        pl.BlockSpec((R, 128), lambda i: (i, 0)),
        pl.BlockSpec((128, 128), lambda i: (0, 0)),
    ],
    out_specs=[
        pl.BlockSpec((R, 128), lambda i: (i, 0)),
        pl.BlockSpec((R, 1), lambda i: (i, 0)),
    ],
    out_shape=[
        jax.ShapeDtypeStruct((N, 128), jnp.float32),
        jax.ShapeDtypeStruct((N, 1), jnp.float32),
    ],
)


def _b2_body(dis_ref, acc_ref, mp1_ref, a1_ref, b1_ref, w2_ref,
             h1_ref, mp2_ref):
    sdis = dis_ref[...]
    conv = sdis * (acc_ref[0] + acc_ref[1] + mp1_ref[...])
    h1 = jnp.maximum(conv * a1_ref[...][None, :] + b1_ref[...][None, :], 0.0)
    h1_ref[...] = h1
    mp2_ref[...] = sdis * jnp.dot(h1, w2_ref[...],
                                  preferred_element_type=jnp.float32)


_b2_call = pl.pallas_call(
    _b2_body,
    grid=(N // R,),
    in_specs=[
        pl.BlockSpec((R, 1), lambda i: (i, 0)),
        pl.BlockSpec((NC, R, 128), lambda i: (0, i, 0)),
        pl.BlockSpec((R, 128), lambda i: (i, 0)),
        pl.BlockSpec((128,), lambda i: (0,)),
        pl.BlockSpec((128,), lambda i: (0,)),
        pl.BlockSpec((128, 128), lambda i: (0, 0)),
    ],
    out_specs=[
        pl.BlockSpec((R, 128), lambda i: (i, 0)),
        pl.BlockSpec((R, 128), lambda i: (i, 0)),
    ],
    out_shape=[
        jax.ShapeDtypeStruct((N, 128), jnp.float32),
        jax.ShapeDtypeStruct((N, 128), jnp.float32),
    ],
)


def _b3_body(dis_ref, acc_ref, mp2_ref, h1_ref, a2_ref, b2_ref,
             lng_ref, lnb_ref, wavg_ref, bavg_ref, w3_ref, mp3_ref):
    sdis = dis_ref[...]
    conv = sdis * (acc_ref[0] + acc_ref[1] + mp2_ref[...])
    h2 = jnp.maximum(conv * a2_ref[...][None, :] + b2_ref[...][None, :], 0.0)
    d = (h2 - h1_ref[...]) * 0.5
    mu = jnp.mean(d, axis=1, keepdims=True)
    dc = d - mu
    var = jnp.mean(dc * dc, axis=1, keepdims=True)
    z = dc / jnp.sqrt(var) * lng_ref[...][None, :] + lnb_ref[...][None, :]
    xr = h2 + jnp.dot(z, wavg_ref[...], preferred_element_type=jnp.float32) \
        + bavg_ref[...][None, :]
    mp3_ref[...] = sdis * jnp.dot(xr, w3_ref[...],
                                  preferred_element_type=jnp.float32)


_b3_call = pl.pallas_call(
    _b3_body,
    grid=(N // R,),
    in_specs=[
        pl.BlockSpec((R, 1), lambda i: (i, 0)),
        pl.BlockSpec((NC, R, 128), lambda i: (0, i, 0)),
        pl.BlockSpec((R, 128), lambda i: (i, 0)),
        pl.BlockSpec((R, 128), lambda i: (i, 0)),
        pl.BlockSpec((128,), lambda i: (0,)),
        pl.BlockSpec((128,), lambda i: (0,)),
        pl.BlockSpec((128,), lambda i: (0,)),
        pl.BlockSpec((128,), lambda i: (0,)),
        pl.BlockSpec((128, 128), lambda i: (0, 0)),
        pl.BlockSpec((128,), lambda i: (0,)),
        pl.BlockSpec((128, 128), lambda i: (0, 0)),
    ],
    out_specs=pl.BlockSpec((R, 128), lambda i: (i, 0)),
    out_shape=jax.ShapeDtypeStruct((N, 128), jnp.float32),
)


def _b4_body(dis_ref, acc_ref, mp3_ref, b3_ref, out_ref):
    sdis = dis_ref[...]
    out_ref[...] = sdis * (acc_ref[0][:, :64] + acc_ref[1][:, :64]
                           + mp3_ref[...][:, :64]) + b3_ref[...][None, :]


_b4_call = pl.pallas_call(
    _b4_body,
    grid=(N // R,),
    in_specs=[
        pl.BlockSpec((R, 1), lambda i: (i, 0)),
        pl.BlockSpec((NC, R, 128), lambda i: (0, i, 0)),
        pl.BlockSpec((R, 128), lambda i: (i, 0)),
        pl.BlockSpec((64,), lambda i: (0,)),
    ],
    out_specs=pl.BlockSpec((R, 64), lambda i: (i, 0)),
    out_shape=jax.ShapeDtypeStruct((N, 64), jnp.float32),
)


def kernel(x, adj, W1, b1, g1, be1, W2, b2, g2, be2,
           lng, lnb, Wavg, bavg, W3, b3):
    src = adj[0]
    dst = adj[1]
    srcp = jnp.concatenate([src, _PAD_SRC]).reshape(NW, NT, NB, CH)
    dstp = jnp.concatenate([dst, _PAD_DST]).reshape(NW, NT, NB, CH)
    dstp_deg = jnp.concatenate([dst, _PAD_DSTD]).reshape(NW, NCHD, 128)
    degparts = _deg_call(dstp_deg)
    degsum = (degparts[0] + degparts[1]).reshape(NPD, 1)

    a1 = BNS * g1
    b1e = b1 * a1 + be1
    a2 = BNS * g2
    b2e = b2 * a2 + be2

    mp1, dis = _b1_call(degsum, x, W1)
    acc1 = _scat128(mp1, srcp, dstp)
    h1, mp2 = _b2_call(dis, acc1, mp1, a1, b1e, W2)
    acc2 = _scat128(mp2, srcp, dstp)
    W3p = jnp.pad(W3, ((0, 0), (0, 64)))
    mp3 = _b3_call(dis, acc2, mp2, h1, a2, b2e, lng, lnb, Wavg, bavg, W3p)
    acc3 = _scat128(mp3, srcp, dstp)
    out = _b4_call(dis, acc3, mp3, b3)
    return out


# speedup vs baseline: 1.0852x; 1.0043x over previous
import functools

import jax
import jax.numpy as jnp
import numpy as np
from jax import lax
from jax.experimental import pallas as pl
from jax.experimental.pallas import tpu as pltpu
from jax.experimental.pallas import tpu_sc as plsc

N = 10000
E = 320000
NP = 10016
NC = 2
NS = 16
NW = NC * NS
CH = 120
NB = 3
NCH = 84
NT = NCH // NB
EPT = NW * NCH * CH
NCHD = 79
EPD = NW * NCHD * 128
NPD = 10112
BNS = 1.0 / (1.0 + 1e-5) ** 0.5
R = 5000

_mesh = plsc.VectorSubcoreMesh(core_axis_name="c", subcore_axis_name="s")
_sc_params = pltpu.CompilerParams(needs_layout_passes=False)

_PAD_SRC = np.arange(EPT - E, dtype=np.int32) % N
_PAD_DST = (N + np.arange(EPT - E, dtype=np.int32) % (NP - N)).astype(np.int32)
_PAD_DSTD = (N + np.arange(EPD - E, dtype=np.int32) % (NP - N)).astype(
    np.int32)


def _deg_body(dst_hbm, deg_hbm, dstbuf, degbuf, stage, colbuf, sumbuf):
    c = lax.axis_index("c")
    s = lax.axis_index("s")
    wid = c * NS + s
    pltpu.sync_copy(dst_hbm.at[wid], dstbuf)
    z = jnp.zeros((16,), jnp.float32)

    def zero(i, carry):
        degbuf[pl.ds(i * 16, 16)] = z
        return carry

    lax.fori_loop(0, NPD // 16, zero, 0)
    ones = jnp.ones((16,), jnp.float32)

    def body(g, carry):
        for k in range(8):
            idx = dstbuf[g, pl.ds(k * 16, 16)]
            plsc.addupdate_scatter(degbuf, [idx], ones)
        return carry

    lax.fori_loop(0, NCHD, body, 0)
    pltpu.sync_copy(degbuf, stage.at[s])
    plsc.subcore_barrier()
    cw = 640

    @pl.when(s == 15)
    def _():
        pltpu.sync_copy(stage.at[:, pl.ds(15 * cw, NPD - 15 * cw)],
                        colbuf.at[:, pl.ds(0, NPD - 15 * cw)])

    @pl.when(s < 15)
    def _():
        pltpu.sync_copy(stage.at[:, pl.ds(s * cw, cw)], colbuf)

    ngrp = jnp.where(s < 15, cw // 16, (NPD - 15 * cw) // 16)

    def red(g, carry):
        v = colbuf[0, pl.ds(g * 16, 16)]
        for k in range(1, NS):
            v = v + colbuf[k, pl.ds(g * 16, 16)]
        sumbuf[pl.ds(g * 16, 16)] = v
        return carry

    lax.fori_loop(0, ngrp, red, 0)

    @pl.when(s < 15)
    def _():
        pltpu.sync_copy(sumbuf, deg_hbm.at[c, pl.ds(s * cw, cw)])

    @pl.when(s == 15)
    def _():
        pltpu.sync_copy(sumbuf.at[pl.ds(0, NPD - 15 * cw)],
                        deg_hbm.at[c, pl.ds(15 * cw, NPD - 15 * cw)])


_deg_call = pl.kernel(
    _deg_body,
    out_type=jax.ShapeDtypeStruct((NC, NPD), jnp.float32),
    mesh=_mesh,
    compiler_params=_sc_params,
    scratch_types=[
        pltpu.VMEM((NCHD, 128), jnp.int32),
        pltpu.VMEM((NPD,), jnp.float32),
        pltpu.VMEM_SHARED((NS, NPD), jnp.float32),
        pltpu.VMEM((NS, 640), jnp.float32),
        pltpu.VMEM((640,), jnp.float32),
    ],
)


def _make_scatter(F):
    def body(mp_hbm, src_hbm, dst_hbm, out_hbm, idxbuf, *rest):
        rows = rest[:NB]
        acc = rest[NB]
        gsems = rest[NB + 1:2 * NB + 1]
        isem = rest[2 * NB + 1]
        rows0 = rows[0]
        c = lax.axis_index("c")
        s = lax.axis_index("s")
        wid = c * NS + s

        z = jnp.zeros((16,), jnp.float32)

        def zero(i, carry):
            for k in range(F // 16):
                rows0[i, pl.ds(k * 16, 16)] = z
            return carry

        lax.fori_loop(0, CH, zero, 0)
        base = s * 632

        def _zero_slice(nrows):
            nf = nrows // CH
            rm = nrows - nf * CH
            for k in range(nf):
                pltpu.sync_copy(rows0, acc.at[pl.ds(base + k * CH, CH)])
            if rm:
                pltpu.sync_copy(rows0.at[pl.ds(0, rm)],
                                acc.at[pl.ds(base + nf * CH, rm)])

        @pl.when(s < 15)
        def _():
            _zero_slice(632)

        @pl.when(s == 15)
        def _():
            _zero_slice(536)

        plsc.subcore_barrier()

        BLK = NB * CH
        base_e = wid * NCH * CH
        pltpu.sync_copy(src_hbm.at[pl.ds(base_e, BLK)],
                        idxbuf.at[pl.ds(0, BLK)])
        pltpu.sync_copy(dst_hbm.at[pl.ds(base_e, BLK)],
                        idxbuf.at[pl.ds(2 * BLK, BLK)])
        for j in range(NB - 1):
            pltpu.async_copy(mp_hbm.at[idxbuf.at[pl.ds(j * CH, CH)]],
                             rows[j], gsems[j])

        def macro(t, carry):
            slot = lax.rem(t, 2)
            nslot = 1 - slot
            nxt = t + 1

            @pl.when(nxt < NT)
            def _():
                pltpu.async_copy(src_hbm.at[pl.ds(base_e + nxt * BLK, BLK)],
                                 idxbuf.at[pl.ds(nslot * BLK, BLK)], isem)
                pltpu.async_copy(dst_hbm.at[pl.ds(base_e + nxt * BLK, BLK)],
                                 idxbuf.at[pl.ds(2 * BLK + nslot * BLK, BLK)],
                                 isem)

            for j in range(NB):
                pltpu.make_async_copy(
                    mp_hbm.at[idxbuf.at[pl.ds(slot * BLK + j * CH, CH)]],
                    rows[j], gsems[j]).wait()
                if j == 0:
                    pltpu.async_copy(
                        mp_hbm.at[
                            idxbuf.at[pl.ds(slot * BLK + (NB - 1) * CH, CH)]],
                        rows[NB - 1], gsems[NB - 1])
                elif j == 1:
                    @pl.when(nxt < NT)
                    def _():
                        pltpu.make_async_copy(
                            src_hbm.at[pl.ds(base_e + nxt * BLK, BLK)],
                            idxbuf.at[pl.ds(nslot * BLK, BLK)], isem).wait()
                        pltpu.make_async_copy(
                            dst_hbm.at[pl.ds(base_e + nxt * BLK, BLK)],
                            idxbuf.at[pl.ds(2 * BLK + nslot * BLK, BLK)],
                            isem).wait()
                        pltpu.async_copy(
                            mp_hbm.at[idxbuf.at[pl.ds(nslot * BLK, CH)]],
                            rows[0], gsems[0])
                else:
                    @pl.when(nxt < NT)
                    def _():
                        pltpu.async_copy(
                            mp_hbm.at[
                                idxbuf.at[
                                    pl.ds(nslot * BLK + (j - 1) * CH, CH)]],
                            rows[j - 1], gsems[j - 1])
                pltpu.sync_copy(
                    rows[j],
                    acc.at[idxbuf.at[pl.ds(2 * BLK + slot * BLK + j * CH,
                                           CH)]],
                    add=True)
            return carry

        lax.fori_loop(0, NT, macro, 0)
        plsc.subcore_barrier()

        @pl.when(s < 15)
        def _():
            pltpu.sync_copy(acc.at[pl.ds(base, 632)],
                            out_hbm.at[c, pl.ds(base, 632)])

        @pl.when(s == 15)
        def _():
            pltpu.sync_copy(acc.at[pl.ds(base, 536)],
                            out_hbm.at[c, pl.ds(base, 536)])

    return pl.kernel(
        body,
        out_type=jax.ShapeDtypeStruct((NC, NP, F), jnp.float32),
        mesh=_mesh,
        compiler_params=_sc_params,
        scratch_types=(
            [pltpu.VMEM((4 * NB * CH,), jnp.int32)]
            + [pltpu.VMEM((CH, F), jnp.float32) for _ in range(NB)]
            + [pltpu.VMEM_SHARED((NP, F), jnp.float32)]
            + [pltpu.SemaphoreType.DMA for _ in range(NB + 1)]
        ),
    )


_scat128 = _make_scatter(128)


def _b1_body(deg_ref, x_ref, w_ref, mp_ref, dis_ref):
    dis = lax.rsqrt(1.0 + deg_ref[...])
    dis_ref[...] = dis
    m = jnp.dot(x_ref[...], w_ref[...], preferred_element_type=jnp.float32)
    mp_ref[...] = m * dis


_b1_call = pl.pallas_call(
    _b1_body,
    grid=(N // R,),
    in_specs=[
        pl.BlockSpec((R, 1), lambda i: (i, 0)),
        pl.BlockSpec((R, 128), lambda i: (i, 0)),
        pl.BlockSpec((128, 128), lambda i: (0, 0)),
    ],
    out_specs=[
        pl.BlockSpec((R, 128), lambda i: (i, 0)),
        pl.BlockSpec((R, 1), lambda i: (i, 0)),
    ],
    out_shape=[
        jax.ShapeDtypeStruct((N, 128), jnp.float32),
        jax.ShapeDtypeStruct((N, 1), jnp.float32),
    ],
)


def _b2_body(dis_ref, acc_ref, mp1_ref, a1_ref, b1_ref, w2_ref,
             h1_ref, mp2_ref):
    sdis = dis_ref[...]
    conv = sdis * (acc_ref[0] + acc_ref[1] + mp1_ref[...])
    h1 = jnp.maximum(conv * a1_ref[...][None, :] + b1_ref[...][None, :], 0.0)
    h1_ref[...] = h1
    mp2_ref[...] = sdis * jnp.dot(h1, w2_ref[...],
                                  preferred_element_type=jnp.float32)


_b2_call = pl.pallas_call(
    _b2_body,
    grid=(N // R,),
    in_specs=[
        pl.BlockSpec((R, 1), lambda i: (i, 0)),
        pl.BlockSpec((NC, R, 128), lambda i: (0, i, 0)),
        pl.BlockSpec((R, 128), lambda i: (i, 0)),
        pl.BlockSpec((128,), lambda i: (0,)),
        pl.BlockSpec((128,), lambda i: (0,)),
        pl.BlockSpec((128, 128), lambda i: (0, 0)),
    ],
    out_specs=[
        pl.BlockSpec((R, 128), lambda i: (i, 0)),
        pl.BlockSpec((R, 128), lambda i: (i, 0)),
    ],
    out_shape=[
        jax.ShapeDtypeStruct((N, 128), jnp.float32),
        jax.ShapeDtypeStruct((N, 128), jnp.float32),
    ],
)


def _b3_body(dis_ref, acc_ref, mp2_ref, h1_ref, a2_ref, b2_ref,
             lng_ref, lnb_ref, wavg_ref, bavg_ref, w3_ref, mp3_ref):
    sdis = dis_ref[...]
    conv = sdis * (acc_ref[0] + acc_ref[1] + mp2_ref[...])
    h2 = jnp.maximum(conv * a2_ref[...][None, :] + b2_ref[...][None, :], 0.0)
    d = (h2 - h1_ref[...]) * 0.5
    mu = jnp.mean(d, axis=1, keepdims=True)
    dc = d - mu
    var = jnp.mean(dc * dc, axis=1, keepdims=True)
    z = dc / jnp.sqrt(var) * lng_ref[...][None, :] + lnb_ref[...][None, :]
    xr = h2 + jnp.dot(z, wavg_ref[...], preferred_element_type=jnp.float32) \
        + bavg_ref[...][None, :]
    mp3_ref[...] = sdis * jnp.dot(xr, w3_ref[...],
                                  preferred_element_type=jnp.float32)


_b3_call = pl.pallas_call(
    _b3_body,
    grid=(N // R,),
    in_specs=[
        pl.BlockSpec((R, 1), lambda i: (i, 0)),
        pl.BlockSpec((NC, R, 128), lambda i: (0, i, 0)),
        pl.BlockSpec((R, 128), lambda i: (i, 0)),
        pl.BlockSpec((R, 128), lambda i: (i, 0)),
        pl.BlockSpec((128,), lambda i: (0,)),
        pl.BlockSpec((128,), lambda i: (0,)),
        pl.BlockSpec((128,), lambda i: (0,)),
        pl.BlockSpec((128,), lambda i: (0,)),
        pl.BlockSpec((128, 128), lambda i: (0, 0)),
        pl.BlockSpec((128,), lambda i: (0,)),
        pl.BlockSpec((128, 128), lambda i: (0, 0)),
    ],
    out_specs=pl.BlockSpec((R, 128), lambda i: (i, 0)),
    out_shape=jax.ShapeDtypeStruct((N, 128), jnp.float32),
)


def _b4_body(dis_ref, acc_ref, mp3_ref, b3_ref, out_ref):
    sdis = dis_ref[...]
    out_ref[...] = sdis * (acc_ref[0][:, :64] + acc_ref[1][:, :64]
                           + mp3_ref[...][:, :64]) + b3_ref[...][None, :]


_b4_call = pl.pallas_call(
    _b4_body,
    grid=(N // R,),
    in_specs=[
        pl.BlockSpec((R, 1), lambda i: (i, 0)),
        pl.BlockSpec((NC, R, 128), lambda i: (0, i, 0)),
        pl.BlockSpec((R, 128), lambda i: (i, 0)),
        pl.BlockSpec((64,), lambda i: (0,)),
    ],
    out_specs=pl.BlockSpec((R, 64), lambda i: (i, 0)),
    out_shape=jax.ShapeDtypeStruct((N, 64), jnp.float32),
)


def kernel(x, adj, W1, b1, g1, be1, W2, b2, g2, be2,
           lng, lnb, Wavg, bavg, W3, b3):
    src = adj[0]
    dst = adj[1]
    srcp = jnp.concatenate([src, _PAD_SRC])
    dstp = jnp.concatenate([dst, _PAD_DST])
    dstp_deg = jnp.concatenate([dst, _PAD_DSTD]).reshape(NW, NCHD, 128)
    degparts = _deg_call(dstp_deg)
    degsum = (degparts[0] + degparts[1]).reshape(NPD, 1)

    a1 = BNS * g1
    b1e = b1 * a1 + be1
    a2 = BNS * g2
    b2e = b2 * a2 + be2

    mp1, dis = _b1_call(degsum, x, W1)
    acc1 = _scat128(mp1, srcp, dstp)
    h1, mp2 = _b2_call(dis, acc1, mp1, a1, b1e, W2)
    acc2 = _scat128(mp2, srcp, dstp)
    W3p = jnp.pad(W3, ((0, 0), (0, 64)))
    mp3 = _b3_call(dis, acc2, mp2, h1, a2, b2e, lng, lnb, Wavg, bavg, W3p)
    acc3 = _scat128(mp3, srcp, dstp)
    out = _b4_call(dis, acc3, mp3, b3)
    return out
